# baseline (speedup 1.0000x reference)
import functools

import jax
import jax.numpy as jnp
from jax import lax
from jax.experimental import pallas as pl
from jax.experimental.pallas import tpu as pltpu
from jax.experimental.pallas import tpu_sc as plsc

N_USERS = 1000000
K = 64
BATCH = 16384
TEMPERATURE = 2.0 / 3.0
LIMIT_LOW = -0.1
LIMIT_HIGH = 1.1

_info = plsc.get_sparse_core_info()
_NC, _NS = _info.num_cores, _info.num_subcores
_NW = _NC * _NS
_B_PER_W = BATCH // _NW


@functools.partial(
    pl.kernel,
    out_type=jax.ShapeDtypeStruct((BATCH, K), jnp.float32),
    mesh=plsc.VectorSubcoreMesh(core_axis_name="c", subcore_axis_name="s"),
    scratch_types=[
        pltpu.VMEM((_B_PER_W,), jnp.int32),
        pltpu.VMEM((_B_PER_W, K), jnp.float32),
        pltpu.SemaphoreType.DMA,
    ],
    compiler_params=pltpu.CompilerParams(use_tc_tiling_on_sc=False),
)
def _sc_gather(ids_hbm, table_hbm, out_hbm, idx_v, rows_v, sem):
    wid = lax.axis_index("s") * _NC + lax.axis_index("c")
    base = wid * _B_PER_W
    pltpu.sync_copy(ids_hbm.at[pl.ds(base, _B_PER_W)], idx_v)
    pltpu.async_copy(table_hbm.at[idx_v], rows_v, sem).wait()
    pltpu.sync_copy(rows_v, out_hbm.at[pl.ds(base, _B_PER_W)])


_ROWS_PER_BLK = 2048
_GRID = BATCH // _ROWS_PER_BLK


def _tc_gate_body(a_ref, u_ref, gh_ref, z_ref, pa_ref):
    a = a_ref[...]
    u = u_ref[...]
    pa_ref[...] = jax.nn.sigmoid(a)
    logistic = jnp.log(u) - jnp.log(1.0 - u)
    s = jax.nn.sigmoid((logistic + a) / TEMPERATURE)
    s_bar = s * (LIMIT_HIGH - LIMIT_LOW) + LIMIT_LOW
    z = jnp.clip(s_bar, 0.0, 1.0)
    z_ref[...] = z
    gh_ref[...] = (z > 0.5).astype(jnp.float32)


def _tc_gate(a, u):
    blk = pl.BlockSpec((_ROWS_PER_BLK, K), lambda i: (i, 0))
    out_sds = jax.ShapeDtypeStruct((BATCH, K), jnp.float32)
    return pl.pallas_call(
        _tc_gate_body,
        grid=(_GRID,),
        in_specs=[blk, blk],
        out_specs=[blk, blk, blk],
        out_shape=[out_sds, out_sds, out_sds],
    )(a, u)


def kernel(user_ids, alpha, u):
    a = _sc_gather(user_ids, alpha)
    gate_hard, z, prob_active = _tc_gate(a, u)
    return (gate_hard, z, prob_active)


# speedup vs baseline: 1.5781x; 1.5781x over previous
import functools

import jax
import jax.numpy as jnp
from jax import lax
from jax.experimental import pallas as pl
from jax.experimental.pallas import tpu as pltpu
from jax.experimental.pallas import tpu_sc as plsc

N_USERS = 1000000
K = 64
BATCH = 16384
TEMPERATURE = 2.0 / 3.0
LIMIT_LOW = -0.1
LIMIT_HIGH = 1.1

_info = plsc.get_sparse_core_info()
_NC, _NS = _info.num_cores, _info.num_subcores
_NW = _NC * _NS
_B_PER_W = BATCH // _NW
_N_CHUNKS = _B_PER_W // 16


@functools.partial(
    pl.kernel,
    out_type=jax.ShapeDtypeStruct((BATCH, K), jnp.float32),
    mesh=plsc.VectorSubcoreMesh(core_axis_name="c", subcore_axis_name="s"),
    scratch_types=[
        pltpu.VMEM((_B_PER_W,), jnp.int32),
        pltpu.VMEM((_B_PER_W, K), jnp.float32),
        pltpu.SemaphoreType.DMA,
    ],
    compiler_params=pltpu.CompilerParams(needs_layout_passes=False),
)
def _sc_gather(ids_hbm, table_hbm, out_hbm, ids_v, rows_v, sem):
    wid = lax.axis_index("s") * _NC + lax.axis_index("c")
    base = wid * _B_PER_W
    pltpu.sync_copy(ids_hbm.at[pl.ds(base, _B_PER_W)], ids_v)

    lanes = lax.iota(jnp.int32, 16)

    def chunk_body(c, _):
        ids16 = ids_v[pl.ds(c * 16, 16)]
        descs = []
        for l in range(16):
            uid = jnp.sum(jnp.where(lanes == l, ids16, 0))
            descs.append(pltpu.async_copy(
                table_hbm.at[pl.ds(uid, 1)],
                rows_v.at[pl.ds(c * 16 + l, 1)],
                sem,
            ))
        for d in descs:
            d.wait()
        return _

    lax.fori_loop(0, _N_CHUNKS, chunk_body, 0)
    pltpu.sync_copy(rows_v, out_hbm.at[pl.ds(base, _B_PER_W)])


_ROWS_PER_BLK = 2048
_GRID = BATCH // _ROWS_PER_BLK


def _tc_gate_body(a_ref, u_ref, gh_ref, z_ref, pa_ref):
    a = a_ref[...]
    u = u_ref[...]
    pa_ref[...] = jax.nn.sigmoid(a)
    logistic = jnp.log(u) - jnp.log(1.0 - u)
    s = jax.nn.sigmoid((logistic + a) / TEMPERATURE)
    s_bar = s * (LIMIT_HIGH - LIMIT_LOW) + LIMIT_LOW
    z = jnp.clip(s_bar, 0.0, 1.0)
    z_ref[...] = z
    gh_ref[...] = (z > 0.5).astype(jnp.float32)


def _tc_gate(a, u):
    blk = pl.BlockSpec((_ROWS_PER_BLK, K), lambda i: (i, 0))
    out_sds = jax.ShapeDtypeStruct((BATCH, K), jnp.float32)
    return pl.pallas_call(
        _tc_gate_body,
        grid=(_GRID,),
        in_specs=[blk, blk],
        out_specs=[blk, blk, blk],
        out_shape=[out_sds, out_sds, out_sds],
    )(a, u)


def kernel(user_ids, alpha, u):
    a = _sc_gather(user_ids, alpha)
    gate_hard, z, prob_active = _tc_gate(a, u)
    return (gate_hard, z, prob_active)


# speedup vs baseline: 1.5852x; 1.0045x over previous
import functools

import jax
import jax.numpy as jnp
from jax import lax
from jax.experimental import pallas as pl
from jax.experimental.pallas import tpu as pltpu
from jax.experimental.pallas import tpu_sc as plsc

N_USERS = 1000000
K = 64
BATCH = 16384
TEMPERATURE = 2.0 / 3.0
LIMIT_LOW = -0.1
LIMIT_HIGH = 1.1

_info = plsc.get_sparse_core_info()
_NC, _NS = _info.num_cores, _info.num_subcores
_NW = _NC * _NS
_B_PER_W = BATCH // _NW
_N_CHUNKS = _B_PER_W // 16


@functools.partial(
    pl.kernel,
    out_type=jax.ShapeDtypeStruct((BATCH, K), jnp.float32),
    mesh=plsc.VectorSubcoreMesh(core_axis_name="c", subcore_axis_name="s"),
    scratch_types=[
        pltpu.VMEM((_B_PER_W,), jnp.int32),
        pltpu.VMEM((_B_PER_W, K), jnp.float32),
        pltpu.SemaphoreType.DMA,
    ],
)
def _sc_gather(ids_hbm, table_hbm, out_hbm, ids_v, rows_v, sem):
    wid = lax.axis_index("s") * _NC + lax.axis_index("c")
    base = wid * _B_PER_W
    pltpu.sync_copy(ids_hbm.at[pl.ds(base, _B_PER_W)], ids_v)

    def chunk_body(c, _):
        ids16 = ids_v[pl.ds(c * 16, 16)]
        descs = []
        for l in range(16):
            uid = ids16[l]
            descs.append(pltpu.async_copy(
                table_hbm.at[pl.ds(uid, 1)],
                rows_v.at[pl.ds(c * 16 + l, 1)],
                sem,
            ))
        for d in descs:
            d.wait()
        return _

    lax.fori_loop(0, _N_CHUNKS, chunk_body, 0)
    pltpu.sync_copy(rows_v, out_hbm.at[pl.ds(base, _B_PER_W)])


_ROWS_PER_BLK = 2048
_GRID = BATCH // _ROWS_PER_BLK


def _tc_gate_body(a_ref, u_ref, gh_ref, z_ref, pa_ref):
    a = a_ref[...]
    u = u_ref[...]
    pa_ref[...] = jax.nn.sigmoid(a)
    logistic = jnp.log(u) - jnp.log(1.0 - u)
    s = jax.nn.sigmoid((logistic + a) / TEMPERATURE)
    s_bar = s * (LIMIT_HIGH - LIMIT_LOW) + LIMIT_LOW
    z = jnp.clip(s_bar, 0.0, 1.0)
    z_ref[...] = z
    gh_ref[...] = (z > 0.5).astype(jnp.float32)


def _tc_gate(a, u):
    blk = pl.BlockSpec((_ROWS_PER_BLK, K), lambda i: (i, 0))
    out_sds = jax.ShapeDtypeStruct((BATCH, K), jnp.float32)
    return pl.pallas_call(
        _tc_gate_body,
        grid=(_GRID,),
        in_specs=[blk, blk],
        out_specs=[blk, blk, blk],
        out_shape=[out_sds, out_sds, out_sds],
    )(a, u)


def kernel(user_ids, alpha, u):
    a = _sc_gather(user_ids, alpha)
    gate_hard, z, prob_active = _tc_gate(a, u)
    return (gate_hard, z, prob_active)


# speedup vs baseline: 1.9151x; 1.2081x over previous
import functools

import jax
import jax.numpy as jnp
from jax import lax
from jax.experimental import pallas as pl
from jax.experimental.pallas import tpu as pltpu
from jax.experimental.pallas import tpu_sc as plsc

N_USERS = 1000000
K = 64
BATCH = 16384
TEMPERATURE = 2.0 / 3.0
LIMIT_LOW = -0.1
LIMIT_HIGH = 1.1

_LAST_COL = (N_USERS - 1) // 128
_LAST_BASE = _LAST_COL * 128

_info = plsc.get_sparse_core_info()
_NC, _NS = _info.num_cores, _info.num_subcores
_NW = _NC * _NS
_B_PER_W = BATCH // _NW
_N_CHUNKS = _B_PER_W // 16
_RING = 8


@functools.partial(
    pl.kernel,
    out_type=jax.ShapeDtypeStruct((BATCH, K), jnp.float32),
    mesh=plsc.VectorSubcoreMesh(core_axis_name="c", subcore_axis_name="s"),
    scratch_types=[
        pltpu.VMEM((_B_PER_W + 16,), jnp.int32),
        pltpu.VMEM((2, K, 128), jnp.float32),
        pltpu.VMEM((2, 1, K), jnp.float32),
        pltpu.VMEM((_RING, K), jnp.float32),
        pltpu.SemaphoreType.DMA,
        pltpu.SemaphoreType.DMA,
        pltpu.SemaphoreType.DMA,
    ],
    compiler_params=pltpu.CompilerParams(needs_layout_passes=False),
)
def _sc_gather(ids_hbm, tableT_hbm, tail_hbm, out_hbm, ids_v, bufs_v,
               tails_v, ring_v, fsem0, fsem1, wsem):
    wid = lax.axis_index("s") * _NC + lax.axis_index("c")
    base = wid * _B_PER_W
    pltpu.sync_copy(ids_hbm.at[pl.ds(base, _B_PER_W)], ids_v.at[pl.ds(0, _B_PER_W)])
    ids_v[pl.ds(_B_PER_W, 16)] = jnp.zeros((16,), jnp.int32)

    kvec = lax.iota(jnp.int32, 16)
    fsems = (fsem0, fsem1)

    def fetch(uid, slot):
        col = lax.shift_right_logical(uid, 7)
        sem = fsems[slot]

        @pl.when(col < _LAST_COL)
        def _():
            off = pl.multiple_of(col * 128, 128)
            pltpu.async_copy(tableT_hbm.at[:, pl.ds(off, 128)],
                             bufs_v.at[slot], sem)

        @pl.when(col >= _LAST_COL)
        def _():
            pltpu.async_copy(tail_hbm.at[pl.ds(uid - _LAST_BASE, 1)],
                             tails_v.at[slot], sem)

    def fetch_wait(uid, slot):
        col = lax.shift_right_logical(uid, 7)
        sem = fsems[slot]

        @pl.when(col < _LAST_COL)
        def _():
            pltpu.make_async_copy(tableT_hbm.at[:, pl.ds(0, 128)],
                                  bufs_v.at[slot], sem).wait()

        @pl.when(col >= _LAST_COL)
        def _():
            pltpu.make_async_copy(tail_hbm.at[pl.ds(0, 1)],
                                  tails_v.at[slot], sem).wait()

    def ring_wait():
        pltpu.make_async_copy(ring_v.at[pl.ds(0, 1)],
                              out_hbm.at[pl.ds(0, 1)], wsem).wait()

    first_uid = jnp.maximum(jnp.minimum(ids_v[pl.ds(0, 16)][0], N_USERS - 1), 0)
    fetch(first_uid, 0)

    def chunk_body(c, carry):
        uid_cur = carry
        ids16 = ids_v[pl.ds(c * 16, 16)]
        ids16n = ids_v[pl.ds(c * 16 + 16, 16)]
        for l in range(16):
            i = c * 16 + l
            slot, nslot = l % 2, (l + 1) % 2
            raw_next = ids16[l + 1] if l < 15 else ids16n[0]
            uid_next = jnp.maximum(jnp.minimum(raw_next, N_USERS - 1), 0)
            fetch(uid_next, nslot)
            fetch_wait(uid_cur, slot)
            lane = jnp.bitwise_and(uid_cur, 127)
            lanes16 = jnp.broadcast_to(lane, (16,))
            col_cur = lax.shift_right_logical(uid_cur, 7)
            r = l % _RING

            @pl.when(col_cur < _LAST_COL)
            def _():
                for q in range(4):
                    vals = plsc.load_gather(bufs_v.at[slot],
                                            [kvec + 16 * q, lanes16])
                    ring_v[r, pl.ds(16 * q, 16)] = vals

            @pl.when(col_cur >= _LAST_COL)
            def _():
                for q in range(4):
                    ring_v[r, pl.ds(16 * q, 16)] = \
                        tails_v[slot, 0, pl.ds(16 * q, 16)]
            pltpu.async_copy(ring_v.at[pl.ds(r, 1)],
                             out_hbm.at[pl.ds(base + i, 1)], wsem)

            @pl.when((c > 0) | (l >= _RING))
            def _():
                ring_wait()

            uid_cur = uid_next
        return uid_cur

    lax.fori_loop(0, _N_CHUNKS, chunk_body, first_uid)
    for _ in range(_RING):
        ring_wait()
    last_uid = jnp.maximum(jnp.minimum(jnp.int32(0), N_USERS - 1), 0)
    fetch_wait(last_uid, 0)


_COLS_PER_BLK = 2048
_GRID = BATCH // _COLS_PER_BLK


def _tc_gate_body(a_ref, u_ref, gh_ref, z_ref, pa_ref):
    aT = jnp.transpose(a_ref[...])
    u = u_ref[...]
    pa_ref[...] = jax.nn.sigmoid(aT)
    logistic = jnp.log(u) - jnp.log(1.0 - u)
    s = jax.nn.sigmoid((logistic + aT) / TEMPERATURE)
    s_bar = s * (LIMIT_HIGH - LIMIT_LOW) + LIMIT_LOW
    z = jnp.clip(s_bar, 0.0, 1.0)
    z_ref[...] = z
    gh_ref[...] = (z > 0.5).astype(jnp.float32)


def _tc_gate(a, uT):
    ablk = pl.BlockSpec((_COLS_PER_BLK, K), lambda i: (i, 0))
    tblk = pl.BlockSpec((K, _COLS_PER_BLK), lambda i: (0, i))
    out_sds = jax.ShapeDtypeStruct((K, BATCH), jnp.float32)
    return pl.pallas_call(
        _tc_gate_body,
        grid=(_GRID,),
        in_specs=[ablk, tblk],
        out_specs=[tblk, tblk, tblk],
        out_shape=[out_sds, out_sds, out_sds],
    )(a, uT)


def kernel(user_ids, alpha, u):
    tail = lax.slice(alpha, (_LAST_BASE, 0), (N_USERS, K))
    a = _sc_gather(user_ids, alpha.T, tail)
    ghT, zT, paT = _tc_gate(a, u.T)
    return (ghT.T, zT.T, paT.T)
